# Initial kernel scaffold; baseline (speedup 1.0000x reference)
import jax
import jax.numpy as jnp
from jax.experimental import pallas as pl


def kernel(contexts, t, in_emb, out_emb):
    raise NotImplementedError("write your pallas kernel here")


# speedup vs baseline: 3.1581x; 3.1581x over previous
import functools

import jax
import jax.numpy as jnp
from jax import lax
from jax.experimental import pallas as pl
from jax.experimental.pallas import tpu as pltpu
from jax.experimental.pallas import tpu_sc as plsc

VOCAB = 100000
HIDDEN = 64
B = 4096
C = 50
T = 50

NC = 2
NS = 16
NW = NC * NS
BPW = B // NW

CP = 56
TP = 56
TG = 4
TPAD = TG * 16


def _cbow_body(ctx_hbm, tid_hbm, in_emb_hbm, out_emb_hbm, y_hbm,
               ctx_v, tid_v, crows_v, trows_v, out_v,
               sem_ci, sem_ti, sem_cr, sem_tr, sem_out):
    wid = lax.axis_index("s") * NC + lax.axis_index("c")
    base = wid * BPW
    lane = lax.iota(jnp.int32, 16)

    pltpu.async_copy(ctx_hbm.at[pl.ds(base, BPW)], ctx_v, sem_ci)
    pltpu.async_copy(tid_hbm.at[pl.ds(base, BPW)], tid_v, sem_ti)
    pltpu.make_async_copy(ctx_hbm.at[pl.ds(base, BPW)], ctx_v, sem_ci).wait()
    pltpu.make_async_copy(tid_hbm.at[pl.ds(base, BPW)], tid_v, sem_ti).wait()

    def per_row(i, carry):
        tdst = trows_v.at[pl.ds(0, TP)]
        pltpu.async_copy(in_emb_hbm.at[ctx_v.at[i]], crows_v, sem_cr)
        pltpu.async_copy(out_emb_hbm.at[tid_v.at[i]], tdst, sem_tr)
        pltpu.make_async_copy(in_emb_hbm.at[ctx_v.at[i]], crows_v, sem_cr).wait()
        pltpu.make_async_copy(out_emb_hbm.at[tid_v.at[i]], tdst, sem_tr).wait()

        def pool(c, accs):
            a0, a1, a2, a3 = accs
            a0 = a0 + crows_v[c, pl.ds(0, 16)]
            a1 = a1 + crows_v[c, pl.ds(16, 16)]
            a2 = a2 + crows_v[c, pl.ds(32, 16)]
            a3 = a3 + crows_v[c, pl.ds(48, 16)]
            return (a0, a1, a2, a3)

        z = jnp.zeros((16,), jnp.float32)
        con0, con1, con2, con3 = lax.fori_loop(0, C, pool, (z, z, z, z))

        def sgroup(g, carry2):
            tb = g * 16
            acc = jnp.zeros((16,), jnp.float32)
            for k in range(16):
                tt = tb + k
                p = trows_v[tt, pl.ds(0, 16)] * con0
                p = p + trows_v[tt, pl.ds(16, 16)] * con1
                p = p + trows_v[tt, pl.ds(32, 16)] * con2
                p = p + trows_v[tt, pl.ds(48, 16)] * con3
                for sh in (8, 4, 2, 1):
                    p = p + p.at[lane ^ sh].get(mode="promise_in_bounds")
                acc = jnp.where(lane == k, p, acc)
            out_v[i, pl.ds(tb, 16)] = acc
            return carry2

        lax.fori_loop(0, TG, sgroup, 0)
        return carry

    lax.fori_loop(0, BPW, per_row, 0)

    pltpu.async_copy(out_v, y_hbm.at[pl.ds(base, BPW)], sem_out)
    pltpu.make_async_copy(out_v, y_hbm.at[pl.ds(base, BPW)], sem_out).wait()


@jax.jit
def _cbow_sc(ctx_pad, tid_pad, in_emb, out_emb):
    mesh = plsc.VectorSubcoreMesh(core_axis_name="c", subcore_axis_name="s")
    f = pl.kernel(
        _cbow_body,
        out_type=jax.ShapeDtypeStruct((B, TPAD), jnp.float32),
        mesh=mesh,
        scratch_types=[
            pltpu.VMEM((BPW, CP), jnp.int32),
            pltpu.VMEM((BPW, TP), jnp.int32),
            pltpu.VMEM((CP, HIDDEN), jnp.float32),
            pltpu.VMEM((TPAD, HIDDEN), jnp.float32),
            pltpu.VMEM((BPW, TPAD), jnp.float32),
            pltpu.SemaphoreType.DMA,
            pltpu.SemaphoreType.DMA,
            pltpu.SemaphoreType.DMA,
            pltpu.SemaphoreType.DMA,
            pltpu.SemaphoreType.DMA,
        ],
        compiler_params=pltpu.CompilerParams(use_tc_tiling_on_sc=False),
    )
    return f(ctx_pad, tid_pad, in_emb, out_emb)


def kernel(contexts, t, in_emb, out_emb):
    contexts = contexts.astype(jnp.int32)
    t = t.astype(jnp.int32)
    ctx_pad = jnp.pad(contexts, ((0, 0), (0, CP - C)))
    tid_pad = jnp.pad(t, ((0, 0), (0, TP - T)))
    y = _cbow_sc(ctx_pad, tid_pad, in_emb, out_emb)
    return y[:, :T].reshape(B, 1, T)
